# baseline (speedup 1.0000x reference)
import dataclasses
import functools

import jax
import jax.numpy as jnp
from jax import lax
from jax.experimental import pallas as pl
from jax.experimental.pallas import tpu as pltpu
from jax.experimental.pallas import tpu_sc as plsc

NC = 2
NS = 16
NW = NC * NS
L = 16

J = 32
NCHUNK = 320
PER_W = J * NCHUNK
E_PAD = NW * PER_W

DPACK = 128
HALF = 64


def _make_kernel(VPAD: int):
    mesh = plsc.VectorSubcoreMesh(core_axis_name="c", subcore_axis_name="s")
    cp = pltpu.CompilerParams()
    if "needs_layout_passes" in pltpu.CompilerParams.__dataclass_fields__:
        cp = dataclasses.replace(cp, needs_layout_passes=False)

    @functools.partial(
        pl.kernel,
        compiler_params=cp,
        out_type=jax.ShapeDtypeStruct((NW, 2, NCHUNK // 2, J), jnp.float32),
        mesh=mesh,
        scratch_types=[
            pltpu.VMEM((2, 2 * J), jnp.int32),
            pltpu.VMEM((2, 2 * J, DPACK), jnp.int32),
            pltpu.VMEM((NCHUNK // 2, J), jnp.float32),
            pltpu.VMEM_SHARED((VPAD, DPACK), jnp.int32),
            pltpu.SemaphoreType.DMA,
            pltpu.SemaphoreType.DMA,
            pltpu.SemaphoreType.DMA,
            pltpu.SemaphoreType.DMA,
        ],
    )
    def ip_kernel(z_hbm, idx_hbm, out_hbm, idx_v, buf_v, out_v, z_sh,
                  sem0, sem1, isem0, isem1):
        wid = lax.axis_index("s") * NC + lax.axis_index("c")
        sid = lax.axis_index("s")
        rows = (VPAD // NS) & ~7
        pltpu.sync_copy(z_hbm.at[pl.ds(sid * rows, rows)],
                        z_sh.at[pl.ds(sid * rows, rows)])
        rem = VPAD - rows * NS

        @pl.when(sid == 0)
        def _tail():
            if rem:
                pltpu.sync_copy(z_hbm.at[pl.ds(rows * NS, rem)],
                                z_sh.at[pl.ds(rows * NS, rem)])

        ibase = wid * (NCHUNK * 2 * J)
        sems = (sem0, sem1)
        isems = (isem0, isem1)
        pltpu.sync_copy(idx_hbm.at[pl.ds(ibase, 2 * J)], idx_v.at[0])
        pltpu.async_copy(idx_hbm.at[pl.ds(ibase + 2 * J, 2 * J)],
                         idx_v.at[1], isem1)
        plsc.subcore_barrier()
        pltpu.async_copy(z_sh.at[idx_v.at[0, pl.ds(0, J)]],
                         buf_v.at[0, pl.ds(0, J)], sem0)
        pltpu.async_copy(z_sh.at[idx_v.at[0, pl.ds(J, J)]],
                         buf_v.at[0, pl.ds(J, J)], sem0)

        lanes = lax.iota(jnp.int32, L)

        def compute(lc, b):
            bb = buf_v.at[b]
            for w0 in range(0, J, L):
                ov = jnp.zeros((L,), jnp.float32)
                for i in range(L):
                    w = w0 + i
                    s = None
                    for k in range(HALF // L):
                        ai = bb[w, pl.ds(k * L, L)]
                        bi = bb[J + w, pl.ds(k * L, L)]
                        p = (plsc.bitcast(ai, jnp.bfloat16)
                             * plsc.bitcast(bi, jnp.bfloat16))
                        x, y = plsc.unpack(
                            p, format=plsc.PackFormat.INTERLEAVED)
                        t = x + y
                        s = t if s is None else s + t
                    ov = jnp.where(lanes == i, jnp.sum(s), ov)
                out_v[lc, pl.ds(w0, L)] = ov

        half_n = NCHUNK // 2
        for h in (0, 1):
            @pl.loop(0, half_n, step=2)
            def _chunks(cc):
                for b in (0, 1):
                    lc = cc + b
                    c = h * half_n + lc

                    @pl.when(c + 1 < NCHUNK)
                    def _next_gather():
                        pltpu.make_async_copy(
                            idx_hbm.at[pl.ds(ibase + (c + 1) * 2 * J,
                                             2 * J)],
                            idx_v.at[1 - b], isems[1 - b]).wait()
                        pltpu.async_copy(
                            z_sh.at[idx_v.at[1 - b, pl.ds(0, J)]],
                            buf_v.at[1 - b, pl.ds(0, J)], sems[1 - b])
                        pltpu.async_copy(
                            z_sh.at[idx_v.at[1 - b, pl.ds(J, J)]],
                            buf_v.at[1 - b, pl.ds(J, J)], sems[1 - b])

                    pltpu.make_async_copy(z_sh.at[idx_v.at[b]],
                                          buf_v.at[b], sems[b]).wait()

                    @pl.when(c + 2 < NCHUNK)
                    def _idx_prefetch():
                        pltpu.async_copy(
                            idx_hbm.at[pl.ds(ibase + (c + 2) * 2 * J, 2 * J)],
                            idx_v.at[b], isems[b])

                    compute(lc, b)

            pltpu.sync_copy(out_v, out_hbm.at[wid, h])

    return ip_kernel


def kernel(z, edge_index):
    V, D = z.shape
    E = edge_index.shape[1]
    idx = edge_index.astype(jnp.int32)
    pad = E_PAD - E
    idx = jnp.pad(idx, ((0, 0), (0, pad)))
    src = idx[0].reshape(NW, NCHUNK, J)
    dst = idx[1].reshape(NW, NCHUNK, J)
    comb = jnp.concatenate([src, dst], axis=2)
    vpad_amt = -V % 8
    z16 = jnp.pad(z, ((0, vpad_amt), (0, 0))).astype(jnp.bfloat16)
    z_packed = lax.bitcast_convert_type(
        z16.reshape(V + vpad_amt, HALF, 2), jnp.int32)
    z_dup = jnp.concatenate([z_packed, z_packed], axis=1)
    out = _make_kernel(V + vpad_amt)(z_dup, comb.reshape(-1))
    return out.reshape(E_PAD)[:E]


# speedup vs baseline: 1.2214x; 1.2214x over previous
import dataclasses
import functools

import jax
import jax.numpy as jnp
from jax import lax
from jax.experimental import pallas as pl
from jax.experimental.pallas import tpu as pltpu
from jax.experimental.pallas import tpu_sc as plsc

NC = 2
NS = 16
NW = NC * NS
L = 16

J = 64
NCHUNK = 160
PER_W = J * NCHUNK
E_PAD = NW * PER_W

DPACK = 128
HALF = 64


def _make_kernel(VPAD: int):
    mesh = plsc.VectorSubcoreMesh(core_axis_name="c", subcore_axis_name="s")
    cp = pltpu.CompilerParams()
    if "needs_layout_passes" in pltpu.CompilerParams.__dataclass_fields__:
        cp = dataclasses.replace(cp, needs_layout_passes=False)

    @functools.partial(
        pl.kernel,
        compiler_params=cp,
        out_type=jax.ShapeDtypeStruct((NW, 2, NCHUNK // 2, J), jnp.float32),
        mesh=mesh,
        scratch_types=[
            pltpu.VMEM((2, 2 * J), jnp.int32),
            pltpu.VMEM((2, 2 * J, DPACK), jnp.int32),
            pltpu.VMEM((NCHUNK // 2, J), jnp.float32),
            pltpu.VMEM_SHARED((VPAD, DPACK), jnp.int32),
            pltpu.SemaphoreType.DMA,
            pltpu.SemaphoreType.DMA,
            pltpu.SemaphoreType.DMA,
            pltpu.SemaphoreType.DMA,
        ],
    )
    def ip_kernel(z_hbm, idx_hbm, out_hbm, idx_v, buf_v, out_v, z_sh,
                  sem0, sem1, isem0, isem1):
        wid = lax.axis_index("s") * NC + lax.axis_index("c")
        sid = lax.axis_index("s")
        rows = (VPAD // NS) & ~7
        pltpu.sync_copy(z_hbm.at[pl.ds(sid * rows, rows)],
                        z_sh.at[pl.ds(sid * rows, rows)])
        rem = VPAD - rows * NS

        @pl.when(sid == 0)
        def _tail():
            if rem:
                pltpu.sync_copy(z_hbm.at[pl.ds(rows * NS, rem)],
                                z_sh.at[pl.ds(rows * NS, rem)])

        ibase = wid * (NCHUNK * 2 * J)
        sems = (sem0, sem1)
        isems = (isem0, isem1)
        pltpu.sync_copy(idx_hbm.at[pl.ds(ibase, 2 * J)], idx_v.at[0])
        pltpu.async_copy(idx_hbm.at[pl.ds(ibase + 2 * J, 2 * J)],
                         idx_v.at[1], isem1)
        plsc.subcore_barrier()
        pltpu.async_copy(z_sh.at[idx_v.at[0]], buf_v.at[0], sem0)

        lanes = lax.iota(jnp.int32, L)

        def compute(lc, b):
            bb = buf_v.at[b]
            for w0 in range(0, J, L):
                ov = jnp.zeros((L,), jnp.float32)
                for i in range(L):
                    w = w0 + i
                    s = None
                    for k in range(HALF // L):
                        ai = bb[w, pl.ds(k * L, L)]
                        bi = bb[J + w, pl.ds(k * L, L)]
                        p = (plsc.bitcast(ai, jnp.bfloat16)
                             * plsc.bitcast(bi, jnp.bfloat16))
                        x, y = plsc.unpack(
                            p, format=plsc.PackFormat.INTERLEAVED)
                        t = x + y
                        s = t if s is None else s + t
                    ov = jnp.where(lanes == i, jnp.sum(s), ov)
                out_v[lc, pl.ds(w0, L)] = ov

        half_n = NCHUNK // 2
        for h in (0, 1):
            @pl.loop(0, half_n, step=2)
            def _chunks(cc):
                for b in (0, 1):
                    lc = cc + b
                    c = h * half_n + lc

                    @pl.when(c + 1 < NCHUNK)
                    def _next_gather():
                        pltpu.make_async_copy(
                            idx_hbm.at[pl.ds(ibase + (c + 1) * 2 * J,
                                             2 * J)],
                            idx_v.at[1 - b], isems[1 - b]).wait()
                        pltpu.async_copy(z_sh.at[idx_v.at[1 - b]],
                                         buf_v.at[1 - b], sems[1 - b])

                    pltpu.make_async_copy(z_sh.at[idx_v.at[b]],
                                          buf_v.at[b], sems[b]).wait()

                    @pl.when(c + 2 < NCHUNK)
                    def _idx_prefetch():
                        pltpu.async_copy(
                            idx_hbm.at[pl.ds(ibase + (c + 2) * 2 * J, 2 * J)],
                            idx_v.at[b], isems[b])

                    compute(lc, b)

            pltpu.sync_copy(out_v, out_hbm.at[wid, h])

    return ip_kernel


def kernel(z, edge_index):
    V, D = z.shape
    E = edge_index.shape[1]
    idx = edge_index.astype(jnp.int32)
    pad = E_PAD - E
    idx = jnp.pad(idx, ((0, 0), (0, pad)))
    src = idx[0].reshape(NW, NCHUNK, J)
    dst = idx[1].reshape(NW, NCHUNK, J)
    comb = jnp.concatenate([src, dst], axis=2)
    vpad_amt = -V % 8
    z16 = jnp.pad(z, ((0, vpad_amt), (0, 0))).astype(jnp.bfloat16)
    z_packed = lax.bitcast_convert_type(
        z16.reshape(V + vpad_amt, HALF, 2), jnp.int32)
    z_dup = jnp.concatenate([z_packed, z_packed], axis=1)
    out = _make_kernel(V + vpad_amt)(z_dup, comb.reshape(-1))
    return out.reshape(E_PAD)[:E]
